# baseline (speedup 1.0000x reference)
import functools

import jax
import jax.numpy as jnp
from jax import lax
from jax.experimental import pallas as pl
from jax.experimental.pallas import tpu as pltpu
from jax.experimental.pallas import tpu_sc as plsc

NC = 2
NS = 16
NW = NC * NS
LN2 = 0.6931471805599453


def _ssp(x):
    return jnp.maximum(x, 0.0) + jnp.log(1.0 + jnp.exp(-jnp.abs(x))) - LN2


def _full(shape):
    return pl.BlockSpec(shape, lambda i: (0,) * len(shape))


def _mesh():
    return plsc.VectorSubcoreMesh(
        core_axis_name="c", subcore_axis_name="s",
        num_cores=NC, num_subcores=NS)


def _tc_project(nf, W1s, W1d, b1, blk=2000):
    N, D = nf.shape
    K = W1s.shape[1]

    def body(nf_ref, Ws_r, Wd_r, b1_r, ps_ref, pd_ref):
        x = nf_ref[...]
        ps_ref[...] = jnp.dot(x, Ws_r[...], preferred_element_type=jnp.float32)
        pd_ref[...] = (jnp.dot(x, Wd_r[...], preferred_element_type=jnp.float32)
                       + b1_r[...])

    return pl.pallas_call(
        body,
        grid=(N // blk,),
        in_specs=[pl.BlockSpec((blk, D), lambda i: (i, 0)),
                  _full(W1s.shape), _full(W1d.shape), _full(b1.shape)],
        out_specs=[pl.BlockSpec((blk, K), lambda i: (i, 0)),
                   pl.BlockSpec((blk, K), lambda i: (i, 0))],
        out_shape=[jax.ShapeDtypeStruct((N, K), jnp.float32),
                   jax.ShapeDtypeStruct((N, K), jnp.float32)],
    )(nf, W1s, W1d, b1)


def _sc_gather(Ps, Pd, src, dst, CH=128):
    E = src.shape[0]
    K = Ps.shape[1]
    total = E // CH
    base_n = total // NW
    extra = total - base_n * NW

    @functools.partial(
        pl.kernel,
        out_type=jax.ShapeDtypeStruct((E, K), jnp.float32),
        mesh=_mesh(),
        compiler_params=pltpu.CompilerParams(use_tc_tiling_on_sc=False),
        scratch_types=[
            pltpu.VMEM((CH,), jnp.int32), pltpu.VMEM((CH,), jnp.int32),
            pltpu.VMEM((CH,), jnp.int32), pltpu.VMEM((CH,), jnp.int32),
            pltpu.VMEM((CH, K), jnp.float32), pltpu.VMEM((CH, K), jnp.float32),
            pltpu.VMEM((CH, K), jnp.float32), pltpu.VMEM((CH, K), jnp.float32),
            pltpu.SemaphoreType.DMA, pltpu.SemaphoreType.DMA,
            pltpu.SemaphoreType.DMA, pltpu.SemaphoreType.DMA,
        ],
    )
    def k(ps_hbm, pd_hbm, src_hbm, dst_hbm, out_hbm,
          is0, is1, id0, id1, ra0, ra1, rb0, rb1, sg0, sg1, sw0, sw1):
        wid = lax.axis_index("s") * NC + lax.axis_index("c")
        my_n = jnp.where(wid < extra, base_n + 1, base_n)
        isv, idv = (is0, is1), (id0, id1)
        ra, rb = (ra0, ra1), (rb0, rb1)
        sg, sw = (sg0, sg1), (sw0, sw1)

        def start_stage(j, b):
            @pl.when(j < my_n)
            def _():
                @pl.when(j >= 2)
                def _():
                    pltpu.make_async_copy(ra[b], out_hbm.at[pl.ds(0, CH)],
                                          sw[b]).wait()
                off = pl.multiple_of((wid + j * NW) * CH, CH)
                pltpu.sync_copy(src_hbm.at[pl.ds(off, CH)], isv[b])
                pltpu.sync_copy(dst_hbm.at[pl.ds(off, CH)], idv[b])
                pltpu.async_copy(ps_hbm.at[isv[b]], ra[b], sg[b])
                pltpu.async_copy(pd_hbm.at[idv[b]], rb[b], sg[b])

        def compute_stage(j, b):
            @pl.when((j >= 0) & (j < my_n))
            def _():
                pltpu.make_async_copy(ps_hbm.at[isv[b]], ra[b], sg[b]).wait()
                pltpu.make_async_copy(pd_hbm.at[idv[b]], rb[b], sg[b]).wait()

                def addrows(r8, carry):
                    for r0 in range(8):
                        for t in range(K // 16):
                            sl = pl.ds(t * 16, 16)
                            plsc.addupdate(ra[b].at[r8 * 8 + r0, sl],
                                           rb[b][r8 * 8 + r0, sl])
                    return carry

                lax.fori_loop(0, CH // 8, addrows, 0)
                off = pl.multiple_of((wid + j * NW) * CH, CH)
                pltpu.async_copy(ra[b], out_hbm.at[pl.ds(off, CH)], sw[b])

        def body(jj, carry):
            for b in (0, 1):
                j = 2 * jj + b
                start_stage(j, b)
                compute_stage(j - 1, 1 - b)
            return carry

        lax.fori_loop(0, (base_n + 1) // 2 + 1, body, 0)
        pltpu.make_async_copy(ra0, out_hbm.at[pl.ds(0, CH)], sw0).wait()
        pltpu.make_async_copy(ra1, out_hbm.at[pl.ds(0, CH)], sw1).wait()

    return k(Ps, Pd, src, dst)


def _tc_edge_mlp(u_pre, ef, W1e, W2, b2, We1, be1, We2, be2, en_buf, boff,
                 blk=1600):
    E2, R = ef.shape
    K = u_pre.shape[1]
    D = We2.shape[1]
    ET = en_buf.shape[0]

    def body(up_ref, ef_ref, W1e_r, W2_r, b2_r, We1_r, be1_r, We2_r, be2_r,
             enb_ref, en_ref, he_ref):
        u = _ssp(up_ref[...]
                 + jnp.dot(ef_ref[...], W1e_r[...],
                           preferred_element_type=jnp.float32))
        en = jnp.dot(u, W2_r[...], preferred_element_type=jnp.float32) + b2_r[...]
        en_ref[...] = en
        t = _ssp(jnp.dot(en, We1_r[...], preferred_element_type=jnp.float32)
                 + be1_r[...])
        he_ref[...] = (jnp.dot(t, We2_r[...], preferred_element_type=jnp.float32)
                       + be2_r[...])

    return pl.pallas_call(
        body,
        grid=(E2 // blk,),
        in_specs=[
            pl.BlockSpec((blk, K), lambda i: (i, 0)),
            pl.BlockSpec((blk, R), lambda i: (i, 0)),
            _full(W1e.shape), _full(W2.shape), _full(b2.shape),
            _full(We1.shape), _full(be1.shape), _full(We2.shape),
            _full(be2.shape),
            pl.BlockSpec((blk, R), lambda i: (boff + i, 0)),
        ],
        out_specs=[
            pl.BlockSpec((blk, R), lambda i: (boff + i, 0)),
            pl.BlockSpec((blk, D), lambda i: (i, 0)),
        ],
        out_shape=[
            jax.ShapeDtypeStruct((ET, R), jnp.float32),
            jax.ShapeDtypeStruct((E2, D), jnp.float32),
        ],
        input_output_aliases={9: 0},
    )(u_pre, ef, W1e, W2, b2, We1, be1, We2, be2, en_buf)


def _tc_remap(dst2, HALF):
    RN, RL = dst2.shape

    def body(d_ref, o_ref):
        v = d_ref[...]
        pad = HALF + (v & 15)
        o_ref[0] = jnp.where(v < HALF, v, pad)
        v1 = v - HALF
        o_ref[1] = jnp.where(v1 >= 0, v1, pad)

    return pl.pallas_call(
        body,
        in_specs=[pl.BlockSpec((RN, RL), lambda: (0, 0))],
        out_specs=pl.BlockSpec((2, RN, RL), lambda: (0, 0, 0)),
        out_shape=jax.ShapeDtypeStruct((2, RN, RL), jnp.int32),
    )(dst2)


def _sc_scatter(he, dl, zeros, N):
    E, D = he.shape
    CH = dl.shape[2]
    HALF = N // NC
    ACC = zeros.shape[0]
    total = E // CH
    base_n = total // NS
    extra = total - base_n * NS
    DR = 1568

    @functools.partial(
        pl.kernel,
        out_type=jax.ShapeDtypeStruct((N, D), jnp.float32),
        mesh=_mesh(),
        compiler_params=pltpu.CompilerParams(use_tc_tiling_on_sc=False),
        scratch_types=[
            pltpu.VMEM((CH,), jnp.int32), pltpu.VMEM((CH,), jnp.int32),
            pltpu.VMEM((CH, D), jnp.float32), pltpu.VMEM((CH, D), jnp.float32),
            pltpu.VMEM_SHARED((ACC, D), jnp.float32),
            pltpu.SemaphoreType.DMA, pltpu.SemaphoreType.DMA,
            pltpu.SemaphoreType.DMA, pltpu.SemaphoreType.DMA,
        ],
    )
    def k(he_hbm, dl_hbm, z_hbm, out_hbm,
          iv0, iv1, rv0, rv1, acc, si0, si1, ss0, ss1):
        c = lax.axis_index("c")
        s = lax.axis_index("s")
        node0 = c * HALF
        my_n = jnp.where(s < extra, base_n + 1, base_n)
        iv, rv = (iv0, iv1), (rv0, rv1)
        si, ss = (si0, si1), (ss0, ss1)
        z0 = pl.multiple_of(s * DR, 8)

        @pl.when(s < NS - 1)
        def _():
            pltpu.sync_copy(z_hbm.at[pl.ds(z0, DR)], acc.at[pl.ds(z0, DR)])

        @pl.when(s == NS - 1)
        def _():
            zl = pl.multiple_of((NS - 1) * DR, 8)
            n = ACC - (NS - 1) * DR
            pltpu.sync_copy(z_hbm.at[pl.ds(zl, n)], acc.at[pl.ds(zl, n)])

        plsc.subcore_barrier()

        def start_stage(j, b):
            @pl.when(j < my_n)
            def _():
                @pl.when(j >= 2)
                def _():
                    pltpu.make_async_copy(rv[b], acc.at[iv[b]], ss[b]).wait()
                row = s + j * NS
                off = pl.multiple_of(row * CH, CH)
                pltpu.async_copy(dl_hbm.at[c, row], iv[b], si[b])
                pltpu.async_copy(he_hbm.at[pl.ds(off, CH)], rv[b], si[b])

        def compute_stage(j, b):
            @pl.when((j >= 0) & (j < my_n))
            def _():
                row = s + j * NS
                pltpu.make_async_copy(dl_hbm.at[c, row], iv[b], si[b]).wait()
                pltpu.make_async_copy(he_hbm.at[pl.ds(0, CH)], rv[b],
                                      si[b]).wait()
                pltpu.async_copy(rv[b], acc.at[iv[b]], ss[b], add=True)

        def body(jj, carry):
            for b in (0, 1):
                j = 2 * jj + b
                start_stage(j, b)
                compute_stage(j - 1, 1 - b)
            return carry

        lax.fori_loop(0, (base_n + 1) // 2 + 1, body, 0)
        pltpu.make_async_copy(rv0, acc.at[iv0], ss0).wait()
        pltpu.make_async_copy(rv1, acc.at[iv1], ss1).wait()
        plsc.subcore_barrier()

        @pl.when(s < NS - 1)
        def _():
            pltpu.sync_copy(acc.at[pl.ds(z0, DR)],
                            out_hbm.at[pl.ds(node0 + z0, DR)])

        @pl.when(s == NS - 1)
        def _():
            zl = pl.multiple_of((NS - 1) * DR, 8)
            n = HALF - (NS - 1) * DR
            pltpu.sync_copy(acc.at[pl.ds(zl, n)],
                            out_hbm.at[pl.ds(node0 + zl, n)])

    return k(he, dl, zeros)


def _tc_node_mlp(node_feats, aggA, aggB, Wa, ba, Wb, bb, blk=2000):
    N, D = node_feats.shape

    def body(nf_ref, ga_ref, gb_ref, Wa_r, ba_r, Wb_r, bb_r, out_ref):
        g = ga_ref[...] + gb_ref[...]
        t = _ssp(jnp.dot(g, Wa_r[...],
                         preferred_element_type=jnp.float32) + ba_r[...])
        out_ref[...] = (nf_ref[...] + bb_r[...]
                        + jnp.dot(t, Wb_r[...],
                                  preferred_element_type=jnp.float32))

    return pl.pallas_call(
        body,
        grid=(N // blk,),
        in_specs=[
            pl.BlockSpec((blk, D), lambda i: (i, 0)),
            pl.BlockSpec((blk, D), lambda i: (i, 0)),
            pl.BlockSpec((blk, D), lambda i: (i, 0)),
            _full(Wa.shape), _full(ba.shape), _full(Wb.shape), _full(bb.shape),
        ],
        out_specs=pl.BlockSpec((blk, D), lambda i: (i, 0)),
        out_shape=jax.ShapeDtypeStruct((N, D), jnp.float32),
    )(node_feats, aggA, aggB, Wa, ba, Wb, bb)


def kernel(node_feats, edge_feats, edge_index, Weu1, beu1, Weu2, beu2,
           Wn1, bn1, We1, be1, We2, be2, Wn2a, bn2a, Wn2b, bn2b):
    N, D = node_feats.shape
    E = edge_feats.shape[0]

    Ps, Pd = _tc_project(node_feats, Weu1[:D], Weu1[D:2 * D],
                         beu1.reshape(1, -1))
    src = edge_index[0]
    dst = edge_index[1]
    E2 = E // 2
    zeros = jnp.zeros((N // NC + 16, D), jnp.float32)

    srcs = (src[:E2], src[E2:])
    dsts = (dst[:E2], dst[E2:])
    dls = tuple(_tc_remap(d.reshape(E2 // 64, 64), N // NC) for d in dsts)

    u0 = _sc_gather(Ps, Pd, srcs[0], dsts[0])
    u1 = _sc_gather(Ps, Pd, srcs[1], dsts[1])

    R = edge_feats.shape[1]
    en_buf = jnp.zeros((E, R), jnp.float32)
    blk = 1600
    en_buf, he0 = _tc_edge_mlp(
        u0, edge_feats[:E2], Weu1[2 * D:], Weu2, beu2.reshape(1, -1),
        We1, be1.reshape(1, -1), We2, be2.reshape(1, -1), en_buf, 0, blk=blk)
    aggA = _sc_scatter(he0, dls[0], zeros, N)
    edge_new, he1 = _tc_edge_mlp(
        u1, edge_feats[E2:], Weu1[2 * D:], Weu2, beu2.reshape(1, -1),
        We1, be1.reshape(1, -1), We2, be2.reshape(1, -1), en_buf, E2 // blk,
        blk=blk)
    aggB = _sc_scatter(he1, dls[1], zeros, N)
    node_out = _tc_node_mlp(node_feats, aggA, aggB,
                            Wn2a, bn2a.reshape(1, -1),
                            Wn2b, bn2b.reshape(1, -1))
    return (node_out, edge_new)


# speedup vs baseline: 1.1010x; 1.1010x over previous
import functools

import jax
import jax.numpy as jnp
from jax import lax
from jax.experimental import pallas as pl
from jax.experimental.pallas import tpu as pltpu
from jax.experimental.pallas import tpu_sc as plsc

NC = 2
NS = 16
NW = NC * NS
LN2 = 0.6931471805599453


def _ssp(x):
    return jnp.maximum(x, 0.0) + jnp.log(1.0 + jnp.exp(-jnp.abs(x))) - LN2


def _full(shape):
    return pl.BlockSpec(shape, lambda i: (0,) * len(shape))


def _mesh():
    return plsc.VectorSubcoreMesh(
        core_axis_name="c", subcore_axis_name="s",
        num_cores=NC, num_subcores=NS)


def _tc_project(nf, W1s, W1d, b1, blk=2000):
    N, D = nf.shape
    K = W1s.shape[1]

    def body(nf_ref, Ws_r, Wd_r, b1_r, ps_ref, pd_ref):
        x = nf_ref[...]
        ps_ref[...] = jnp.dot(x, Ws_r[...], preferred_element_type=jnp.float32)
        pd_ref[...] = (jnp.dot(x, Wd_r[...], preferred_element_type=jnp.float32)
                       + b1_r[...])

    return pl.pallas_call(
        body,
        grid=(N // blk,),
        in_specs=[pl.BlockSpec((blk, D), lambda i: (i, 0)),
                  _full(W1s.shape), _full(W1d.shape), _full(b1.shape)],
        out_specs=[pl.BlockSpec((blk, K), lambda i: (i, 0)),
                   pl.BlockSpec((blk, K), lambda i: (i, 0))],
        out_shape=[jax.ShapeDtypeStruct((N, K), jnp.float32),
                   jax.ShapeDtypeStruct((N, K), jnp.float32)],
    )(nf, W1s, W1d, b1)


def _sc_gather(Ps, Pd, src, dst, CH=128):
    E = src.shape[0]
    K = Ps.shape[1]
    total = E // CH
    base_n = total // NW
    extra = total - base_n * NW

    @functools.partial(
        pl.kernel,
        out_type=jax.ShapeDtypeStruct((E, K), jnp.float32),
        mesh=_mesh(),
        compiler_params=pltpu.CompilerParams(use_tc_tiling_on_sc=False),
        scratch_types=[
            pltpu.VMEM((CH,), jnp.int32), pltpu.VMEM((CH,), jnp.int32),
            pltpu.VMEM((CH,), jnp.int32), pltpu.VMEM((CH,), jnp.int32),
            pltpu.VMEM((CH, K), jnp.float32), pltpu.VMEM((CH, K), jnp.float32),
            pltpu.VMEM((CH, K), jnp.float32), pltpu.VMEM((CH, K), jnp.float32),
            pltpu.SemaphoreType.DMA, pltpu.SemaphoreType.DMA,
            pltpu.SemaphoreType.DMA, pltpu.SemaphoreType.DMA,
        ],
    )
    def k(ps_hbm, pd_hbm, src_hbm, dst_hbm, out_hbm,
          is0, is1, id0, id1, ra0, ra1, rb0, rb1, sg0, sg1, sw0, sw1):
        wid = lax.axis_index("s") * NC + lax.axis_index("c")
        my_n = jnp.where(wid < extra, base_n + 1, base_n)
        isv, idv = (is0, is1), (id0, id1)
        ra, rb = (ra0, ra1), (rb0, rb1)
        sg, sw = (sg0, sg1), (sw0, sw1)

        def start_stage(j, b):
            @pl.when(j < my_n)
            def _():
                @pl.when(j >= 2)
                def _():
                    pltpu.make_async_copy(ra[b], out_hbm.at[pl.ds(0, CH)],
                                          sw[b]).wait()
                off = pl.multiple_of((wid + j * NW) * CH, CH)
                pltpu.sync_copy(src_hbm.at[pl.ds(off, CH)], isv[b])
                pltpu.sync_copy(dst_hbm.at[pl.ds(off, CH)], idv[b])
                pltpu.async_copy(ps_hbm.at[isv[b]], ra[b], sg[b])
                pltpu.async_copy(pd_hbm.at[idv[b]], rb[b], sg[b])

        def compute_stage(j, b):
            @pl.when((j >= 0) & (j < my_n))
            def _():
                pltpu.make_async_copy(ps_hbm.at[isv[b]], ra[b], sg[b]).wait()
                pltpu.make_async_copy(pd_hbm.at[idv[b]], rb[b], sg[b]).wait()

                def addrows(r8, carry):
                    for r0 in range(8):
                        for t in range(K // 16):
                            sl = pl.ds(t * 16, 16)
                            plsc.addupdate(ra[b].at[r8 * 8 + r0, sl],
                                           rb[b][r8 * 8 + r0, sl])
                    return carry

                lax.fori_loop(0, CH // 8, addrows, 0)
                off = pl.multiple_of((wid + j * NW) * CH, CH)
                pltpu.async_copy(ra[b], out_hbm.at[pl.ds(off, CH)], sw[b])

        def body(jj, carry):
            for b in (0, 1):
                j = 2 * jj + b
                start_stage(j, b)
                compute_stage(j - 1, 1 - b)
            return carry

        lax.fori_loop(0, (base_n + 1) // 2 + 1, body, 0)
        pltpu.make_async_copy(ra0, out_hbm.at[pl.ds(0, CH)], sw0).wait()
        pltpu.make_async_copy(ra1, out_hbm.at[pl.ds(0, CH)], sw1).wait()

    return k(Ps, Pd, src, dst)


def _tc_edge_mlp(u_pre, ef, W1e, W2, b2, We1, be1, We2, be2, en_buf, boff,
                 ET, blk=1600):
    E2, R = ef.shape
    K = u_pre.shape[1]
    D = We2.shape[1]

    def body(up_ref, ef_ref, W1e_r, W2_r, b2_r, We1_r, be1_r, We2_r, be2_r,
             *rest):
        en_ref, he_ref = rest[-2], rest[-1]
        u = _ssp(up_ref[...]
                 + jnp.dot(ef_ref[...], W1e_r[...],
                           preferred_element_type=jnp.float32))
        en = jnp.dot(u, W2_r[...], preferred_element_type=jnp.float32) + b2_r[...]
        en_ref[...] = en
        t = _ssp(jnp.dot(en, We1_r[...], preferred_element_type=jnp.float32)
                 + be1_r[...])
        he_ref[...] = (jnp.dot(t, We2_r[...], preferred_element_type=jnp.float32)
                       + be2_r[...])

    in_specs = [
        pl.BlockSpec((blk, K), lambda i: (i, 0)),
        pl.BlockSpec((blk, R), lambda i: (i, 0)),
        _full(W1e.shape), _full(W2.shape), _full(b2.shape),
        _full(We1.shape), _full(be1.shape), _full(We2.shape),
        _full(be2.shape),
    ]
    args = [u_pre, ef, W1e, W2, b2, We1, be1, We2, be2]
    aliases = {}
    if en_buf is not None:
        in_specs.append(pl.BlockSpec((blk, R), lambda i: (boff + i, 0)))
        args.append(en_buf)
        aliases = {9: 0}
    return pl.pallas_call(
        body,
        grid=(E2 // blk,),
        in_specs=in_specs,
        out_specs=[
            pl.BlockSpec((blk, R), lambda i: (boff + i, 0)),
            pl.BlockSpec((blk, D), lambda i: (i, 0)),
        ],
        out_shape=[
            jax.ShapeDtypeStruct((ET, R), jnp.float32),
            jax.ShapeDtypeStruct((E2, D), jnp.float32),
        ],
        input_output_aliases=aliases,
    )(*args)


def _tc_remap(dst2, HALF):
    RN, RL = dst2.shape

    def body(d_ref, o_ref):
        v = d_ref[...]
        pad = HALF + (v & 15)
        o_ref[0] = jnp.where(v < HALF, v, pad)
        v1 = v - HALF
        o_ref[1] = jnp.where(v1 >= 0, v1, pad)

    return pl.pallas_call(
        body,
        in_specs=[pl.BlockSpec((RN, RL), lambda: (0, 0))],
        out_specs=pl.BlockSpec((2, RN, RL), lambda: (0, 0, 0)),
        out_shape=jax.ShapeDtypeStruct((2, RN, RL), jnp.int32),
    )(dst2)


def _sc_scatter(he, dl, zeros, N):
    E, D = he.shape
    CH = dl.shape[2]
    HALF = N // NC
    ACC = zeros.shape[0]
    total = E // CH
    base_n = total // NS
    extra = total - base_n * NS
    DR = 1568

    @functools.partial(
        pl.kernel,
        out_type=jax.ShapeDtypeStruct((N, D), jnp.float32),
        mesh=_mesh(),
        compiler_params=pltpu.CompilerParams(use_tc_tiling_on_sc=False),
        scratch_types=[
            pltpu.VMEM((CH,), jnp.int32), pltpu.VMEM((CH,), jnp.int32),
            pltpu.VMEM((CH, D), jnp.float32), pltpu.VMEM((CH, D), jnp.float32),
            pltpu.VMEM_SHARED((ACC, D), jnp.float32),
            pltpu.SemaphoreType.DMA, pltpu.SemaphoreType.DMA,
            pltpu.SemaphoreType.DMA, pltpu.SemaphoreType.DMA,
        ],
    )
    def k(he_hbm, dl_hbm, z_hbm, out_hbm,
          iv0, iv1, rv0, rv1, acc, si0, si1, ss0, ss1):
        c = lax.axis_index("c")
        s = lax.axis_index("s")
        node0 = c * HALF
        my_n = jnp.where(s < extra, base_n + 1, base_n)
        iv, rv = (iv0, iv1), (rv0, rv1)
        si, ss = (si0, si1), (ss0, ss1)
        z0 = pl.multiple_of(s * DR, 8)

        @pl.when(s < NS - 1)
        def _():
            pltpu.sync_copy(z_hbm.at[pl.ds(z0, DR)], acc.at[pl.ds(z0, DR)])

        @pl.when(s == NS - 1)
        def _():
            zl = pl.multiple_of((NS - 1) * DR, 8)
            n = ACC - (NS - 1) * DR
            pltpu.sync_copy(z_hbm.at[pl.ds(zl, n)], acc.at[pl.ds(zl, n)])

        plsc.subcore_barrier()

        def start_stage(j, b):
            @pl.when(j < my_n)
            def _():
                @pl.when(j >= 2)
                def _():
                    pltpu.make_async_copy(rv[b], acc.at[iv[b]], ss[b]).wait()
                row = s + j * NS
                off = pl.multiple_of(row * CH, CH)
                pltpu.async_copy(dl_hbm.at[c, row], iv[b], si[b])
                pltpu.async_copy(he_hbm.at[pl.ds(off, CH)], rv[b], si[b])

        def compute_stage(j, b):
            @pl.when((j >= 0) & (j < my_n))
            def _():
                row = s + j * NS
                pltpu.make_async_copy(dl_hbm.at[c, row], iv[b], si[b]).wait()
                pltpu.make_async_copy(he_hbm.at[pl.ds(0, CH)], rv[b],
                                      si[b]).wait()
                pltpu.async_copy(rv[b], acc.at[iv[b]], ss[b], add=True)

        def body(jj, carry):
            for b in (0, 1):
                j = 2 * jj + b
                start_stage(j, b)
                compute_stage(j - 1, 1 - b)
            return carry

        lax.fori_loop(0, (base_n + 1) // 2 + 1, body, 0)
        pltpu.make_async_copy(rv0, acc.at[iv0], ss0).wait()
        pltpu.make_async_copy(rv1, acc.at[iv1], ss1).wait()
        plsc.subcore_barrier()

        @pl.when(s < NS - 1)
        def _():
            pltpu.sync_copy(acc.at[pl.ds(z0, DR)],
                            out_hbm.at[pl.ds(node0 + z0, DR)])

        @pl.when(s == NS - 1)
        def _():
            zl = pl.multiple_of((NS - 1) * DR, 8)
            n = HALF - (NS - 1) * DR
            pltpu.sync_copy(acc.at[pl.ds(zl, n)],
                            out_hbm.at[pl.ds(node0 + zl, n)])

    return k(he, dl, zeros)


def _tc_node_mlp(node_feats, aggA, aggB, Wa, ba, Wb, bb, blk=2000):
    N, D = node_feats.shape

    def body(nf_ref, ga_ref, gb_ref, Wa_r, ba_r, Wb_r, bb_r, out_ref):
        g = ga_ref[...] + gb_ref[...]
        t = _ssp(jnp.dot(g, Wa_r[...],
                         preferred_element_type=jnp.float32) + ba_r[...])
        out_ref[...] = (nf_ref[...] + bb_r[...]
                        + jnp.dot(t, Wb_r[...],
                                  preferred_element_type=jnp.float32))

    return pl.pallas_call(
        body,
        grid=(N // blk,),
        in_specs=[
            pl.BlockSpec((blk, D), lambda i: (i, 0)),
            pl.BlockSpec((blk, D), lambda i: (i, 0)),
            pl.BlockSpec((blk, D), lambda i: (i, 0)),
            _full(Wa.shape), _full(ba.shape), _full(Wb.shape), _full(bb.shape),
        ],
        out_specs=pl.BlockSpec((blk, D), lambda i: (i, 0)),
        out_shape=jax.ShapeDtypeStruct((N, D), jnp.float32),
    )(node_feats, aggA, aggB, Wa, ba, Wb, bb)


def kernel(node_feats, edge_feats, edge_index, Weu1, beu1, Weu2, beu2,
           Wn1, bn1, We1, be1, We2, be2, Wn2a, bn2a, Wn2b, bn2b):
    N, D = node_feats.shape
    E = edge_feats.shape[0]

    Ps, Pd = _tc_project(node_feats, Weu1[:D], Weu1[D:2 * D],
                         beu1.reshape(1, -1))
    src = edge_index[0]
    dst = edge_index[1]
    E2 = E // 2
    zeros = jnp.zeros((N // NC + 16, D), jnp.float32)

    srcs = (src[:E2], src[E2:])
    dsts = (dst[:E2], dst[E2:])
    dls = tuple(_tc_remap(d.reshape(E2 // 64, 64), N // NC) for d in dsts)

    u0 = _sc_gather(Ps, Pd, srcs[0], dsts[0])
    u1 = _sc_gather(Ps, Pd, srcs[1], dsts[1])

    blk = 1600
    en_buf, he0 = _tc_edge_mlp(
        u0, edge_feats[:E2], Weu1[2 * D:], Weu2, beu2.reshape(1, -1),
        We1, be1.reshape(1, -1), We2, be2.reshape(1, -1), None, 0, E, blk=blk)
    aggA = _sc_scatter(he0, dls[0], zeros, N)
    edge_new, he1 = _tc_edge_mlp(
        u1, edge_feats[E2:], Weu1[2 * D:], Weu2, beu2.reshape(1, -1),
        We1, be1.reshape(1, -1), We2, be2.reshape(1, -1), en_buf, E2 // blk,
        E, blk=blk)
    aggB = _sc_scatter(he1, dls[1], zeros, N)
    node_out = _tc_node_mlp(node_feats, aggA, aggB,
                            Wn2a, bn2a.reshape(1, -1),
                            Wn2b, bn2b.reshape(1, -1))
    return (node_out, edge_new)
